# Initial kernel scaffold; baseline (speedup 1.0000x reference)
import jax
import jax.numpy as jnp
from jax.experimental import pallas as pl


def kernel(x_0, x_1, x_2, adjacency_0, adjacency_1, adjacency_2, incidence_1, incidence_2, params):
    raise NotImplementedError("write your pallas kernel here")


# speedup vs baseline: 3.8021x; 3.8021x over previous
import functools

import jax
import jax.numpy as jnp
from jax import lax
from jax.experimental import pallas as pl
from jax.experimental.pallas import tpu as pltpu
from jax.experimental.pallas import tpu_sc as plsc

_N0, _N1, _N2 = 10000, 40000, 5000
_H = 128
_C = 32
_NS = 0.2
_R0, _R1, _R2 = 10240, 40960, 5120
_NW = 32


def _round_up(x, m):
    return (x + m - 1) // m * m


@functools.cache
def _make_agg(e_pad, n_table, r_out, p_passes):
    F = _H // p_passes
    EBLK = 64 if p_passes == 1 else 128
    EPW = e_pad // _NW
    NB = EPW // EBLK
    assert EPW % EBLK == 0
    rows_pt = r_out // 16
    ZR = 32 if p_passes == 1 else 128
    OB = 64
    assert rows_pt % ZR == 0 and rows_pt % OB == 0

    mesh = plsc.VectorSubcoreMesh(core_axis_name="c", subcore_axis_name="s")

    def body(*refs):
        tabs = refs[:p_passes]
        (gidx, sidx, ag_hbm, as_hbm, acc_out, d_out,
         gbuf, sbuf, wbuf, av1, av2, rows, zrow, obuf, dbuf,
         acc_sh, d_sh, sem, sem1, sem2) = refs[p_passes:]

        cid = lax.axis_index("c")
        sid = lax.axis_index("s")
        wid = cid * 16 + sid
        r0t = sid * rows_pt

        zero16 = jnp.zeros((16,), jnp.float32)
        for j in range(ZR):
            for f in range(F // 16):
                zrow[j, pl.ds(f * 16, 16)] = zero16
        for j in range(rows_pt // 16):
            dbuf[pl.ds(j * 16, 16)] = zero16

        for q in range(p_passes):
            @pl.loop(0, rows_pt // ZR)
            def _zero(k):
                pltpu.sync_copy(zrow, acc_sh.at[pl.ds(r0t + k * ZR, ZR)])
            if q == 0:
                pltpu.sync_copy(dbuf, d_sh.at[pl.ds(r0t, rows_pt)])
            plsc.subcore_barrier()

            @pl.loop(0, NB)
            def _edges(b):
                off = wid * EPW + b * EBLK
                pltpu.sync_copy(gidx.at[pl.ds(off, EBLK)], gbuf)
                pltpu.sync_copy(sidx.at[pl.ds(off, EBLK)], sbuf)
                cp0 = pltpu.async_copy(tabs[q].at[gbuf], rows, sem)
                cp1 = pltpu.async_copy(ag_hbm.at[gbuf], av1, sem1)
                cp2 = pltpu.async_copy(as_hbm.at[sbuf], av2, sem2)
                cp1.wait()
                cp2.wait()
                cp0.wait()
                for j in range(EBLK // 16):
                    e = (av1[pl.ds(j * 16, 16)] + av2[pl.ds(j * 16, 16)])
                    e = jnp.where(e > 0, e, _NS * e)
                    w16 = jnp.exp(e)
                    if q == 0:
                        wbuf[pl.ds(j * 16, 16)] = w16
                    for l in range(16):
                        i = j * 16 + l
                        w = w16[l]
                        for f in range(F // 16):
                            rows[i, pl.ds(f * 16, 16)] = (
                                rows[i, pl.ds(f * 16, 16)] * w)
                if q == 0:
                    pltpu.sync_copy(wbuf, d_sh.at[sbuf], add=True)
                pltpu.sync_copy(rows, acc_sh.at[sbuf], add=True)

            plsc.subcore_barrier()

            @pl.loop(0, rows_pt // OB)
            def _wout(k):
                pltpu.sync_copy(acc_sh.at[pl.ds(r0t + k * OB, OB)], obuf)
                pltpu.sync_copy(obuf,
                                acc_out.at[cid, pl.ds(r0t + k * OB, OB), q])
            if q == 0:
                pltpu.sync_copy(d_sh.at[pl.ds(r0t, rows_pt)], dbuf)
                pltpu.sync_copy(
                    dbuf, d_out.at[pl.ds(cid * r_out + r0t, rows_pt)])
            plsc.subcore_barrier()

    out_type = (jax.ShapeDtypeStruct((2, r_out, p_passes, F), jnp.float32),
                jax.ShapeDtypeStruct((2 * r_out,), jnp.float32))
    scratch = (
        pltpu.VMEM((EBLK,), jnp.int32),
        pltpu.VMEM((EBLK,), jnp.int32),
        pltpu.VMEM((EBLK,), jnp.float32),
        pltpu.VMEM((EBLK,), jnp.float32),
        pltpu.VMEM((EBLK,), jnp.float32),
        pltpu.VMEM((EBLK, F), jnp.float32),
        pltpu.VMEM((ZR, F), jnp.float32),
        pltpu.VMEM((OB, F), jnp.float32),
        pltpu.VMEM((rows_pt,), jnp.float32),
        pltpu.VMEM_SHARED((r_out, F), jnp.float32),
        pltpu.VMEM_SHARED((r_out,), jnp.float32),
        pltpu.SemaphoreType.DMA,
        pltpu.SemaphoreType.DMA,
        pltpu.SemaphoreType.DMA,
    )
    return pl.kernel(body, out_type=out_type, mesh=mesh,
                     scratch_types=scratch,
                     compiler_params=pltpu.CompilerParams(
                         needs_layout_passes=False,
                         use_tc_tiling_on_sc=False))


def _agg(tables, gidx, sidx, ag, as_pad, r_out):
    p = len(tables)
    e_pad = gidx.shape[0]
    fn = _make_agg(e_pad, tables[0].shape[0], r_out, p)
    acc, d = fn(*tables, gidx, sidx, ag, as_pad)
    return acc.reshape(2, r_out, _H), d.reshape(2, r_out)


_BN = 1000


def _proj(x, w):
    n, k = x.shape[0], w.shape[1]

    def kern(x_ref, w_ref, o_ref):
        o_ref[...] = jnp.dot(x_ref[...], w_ref[...],
                             preferred_element_type=jnp.float32)

    return pl.pallas_call(
        kern,
        grid=(n // _BN,),
        in_specs=[pl.BlockSpec((_BN, _H), lambda i: (i, 0)),
                  pl.BlockSpec((_H, k), lambda i: (0, 0))],
        out_specs=pl.BlockSpec((_BN, k), lambda i: (i, 0)),
        out_shape=jax.ShapeDtypeStruct((n, k), jnp.float32),
    )(x, w)


def _epi(jobs, n, w, b=None):
    k = w.shape[1]
    nj = len(jobs)

    def kern(*refs):
        o_ref = refs[-1]
        w_ref = refs[2 * nj]
        parts = None
        for j in range(nj):
            acc = refs[2 * j][...]
            d = refs[2 * j + 1][...]
            dt = d[:, 0] + d[:, 1] + 1e-16
            m = (acc[0] + acc[1]) / dt[:, None]
            parts = m if parts is None else parts + m
        xblk = jnp.maximum(parts, 0.0)
        out = jnp.dot(xblk, w_ref[...], preferred_element_type=jnp.float32)
        if b is not None:
            out = out + refs[2 * nj + 1][...]
        o_ref[...] = out

    in_specs = []
    args = []
    for acc, d in jobs:
        in_specs.append(pl.BlockSpec((2, _BN, _H), lambda i: (0, i, 0)))
        in_specs.append(pl.BlockSpec((_BN, 2), lambda i: (i, 0)))
        args += [acc, d.T]
    in_specs.append(pl.BlockSpec((_H, k), lambda i: (0, 0)))
    args.append(w)
    if b is not None:
        in_specs.append(pl.BlockSpec((1, k), lambda i: (0, 0)))
        args.append(b.reshape(1, k))

    return pl.pallas_call(
        kern,
        grid=(n // _BN,),
        in_specs=in_specs,
        out_specs=pl.BlockSpec((_BN, k), lambda i: (i, 0)),
        out_shape=jax.ShapeDtypeStruct((n, k), jnp.float32),
    )(*args)


def _wcat(lp):
    h0, h1, h2 = lp['hbs0'], lp['hbs1'], lp['hbs2']
    b01, b12 = lp['hbns01'], lp['hbns12']
    z = jnp.zeros
    w0 = jnp.concatenate([
        h0['W'], b01['Ws'],
        (h0['W'] @ h0['a_src'])[:, None], (h0['W'] @ h0['a_dst'])[:, None],
        (b01['Ws'] @ b01['a_s'])[:, None], z((_H, 125))], axis=1)
    w1 = jnp.concatenate([
        b01['Wt'], h1['W'], b12['Ws'],
        (b01['Wt'] @ b01['a_t'])[:, None],
        (h1['W'] @ h1['a_src'])[:, None], (h1['W'] @ h1['a_dst'])[:, None],
        (b12['Ws'] @ b12['a_s'])[:, None], z((_H, 124))], axis=1)
    w2 = jnp.concatenate([
        b12['Wt'], h2['W'],
        (b12['Wt'] @ b12['a_t'])[:, None],
        (h2['W'] @ h2['a_src'])[:, None], (h2['W'] @ h2['a_dst'])[:, None],
        z((_H, 125))], axis=1)
    return w0, w1, w2


def _split(m, p):
    f = _H // p
    return tuple(m[:, f * q:f * (q + 1)] for q in range(p))


def _pad_alpha(a, r):
    return jnp.pad(a, (0, r - a.shape[0]))


def _level_jobs(m0, m1, m2, eidx):
    (g_hbs0, s_hbs0, g_01t, s_01t, g_01s, s_01s, g_hbs1, s_hbs1,
     g_12t, s_12t, g_12s, s_12s, g_hbs2, s_hbs2) = eidx

    a_src0, a_dst0, a_s01 = m0[:, 256], m0[:, 257], m0[:, 258]
    a_t01, a_src1, a_dst1, a_s12 = (m1[:, 384], m1[:, 385], m1[:, 386],
                                    m1[:, 387])
    a_t12, a_src2, a_dst2 = m2[:, 256], m2[:, 257], m2[:, 258]

    hbs0 = _agg(_split(m0[:, :128], 2), g_hbs0, s_hbs0,
                a_src0, _pad_alpha(a_dst0, _R0), _R0)
    j01t = _agg(_split(m0[:, 128:256], 8), g_01t, s_01t,
                a_s01, _pad_alpha(a_t01, _R1), _R1)
    j01s = _agg(_split(m1[:, :128], 2), g_01s, s_01s,
                a_t01, _pad_alpha(a_s01, _R0), _R0)
    hbs1 = _agg(_split(m1[:, 128:256], 8), g_hbs1, s_hbs1,
                a_src1, _pad_alpha(a_dst1, _R1), _R1)
    j12t = _agg((m1[:, 256:384],), g_12t, s_12t,
                a_s12, _pad_alpha(a_t12, _R2), _R2)
    j12s = _agg(_split(m2[:, :128], 8), g_12s, s_12s,
                a_t12, _pad_alpha(a_s12, _R1), _R1)
    hbs2 = _agg((m2[:, 128:256],), g_hbs2, s_hbs2,
                a_src2, _pad_alpha(a_dst2, _R2), _R2)
    return hbs0, j01t, j01s, hbs1, j12t, j12s, hbs2


def _pad_edges(g, s, n_t, unit):
    e = g.shape[0]
    e_pad = _round_up(e, unit)
    return (jnp.pad(g, (0, e_pad - e)),
            jnp.pad(s, (0, e_pad - e), constant_values=n_t))


def kernel(x_0, x_1, x_2, adjacency_0, adjacency_1, adjacency_2,
           incidence_1, incidence_2, params):
    u1, u4 = 2048, 4096
    g_hbs0, s_hbs0 = _pad_edges(adjacency_0[0], adjacency_0[1], _N0, u4)
    g_01t, s_01t = _pad_edges(incidence_1[0], incidence_1[1], _N1, u4)
    g_01s, s_01s = _pad_edges(incidence_1[1], incidence_1[0], _N0, u4)
    g_hbs1, s_hbs1 = _pad_edges(adjacency_1[0], adjacency_1[1], _N1, u4)
    g_12t, s_12t = _pad_edges(incidence_2[0], incidence_2[1], _N2, u1)
    g_12s, s_12s = _pad_edges(incidence_2[1], incidence_2[0], _N1, u4)
    g_hbs2, s_hbs2 = _pad_edges(adjacency_2[0], adjacency_2[1], _N2, u1)
    eidx = (g_hbs0, s_hbs0, g_01t, s_01t, g_01s, s_01s, g_hbs1, s_hbs1,
            g_12t, s_12t, g_12s, s_12s, g_hbs2, s_hbs2)

    w0_l1, w1_l1, w2_l1 = _wcat(params['l1'])
    w0_l2, w1_l2, w2_l2 = _wcat(params['l2'])

    m0 = _proj(x_0, w0_l1)
    m1 = _proj(x_1, w1_l1)
    m2 = _proj(x_2, w2_l1)
    hbs0, j01t, j01s, hbs1, j12t, j12s, hbs2 = _level_jobs(m0, m1, m2, eidx)

    m0 = _epi([hbs0, j01s], _N0, w0_l2)
    m1 = _epi([j01t, hbs1, j12s], _N1, w1_l2)
    m2 = _epi([j12t, hbs2], _N2, w2_l2)

    hbs0, j01t, j01s, hbs1, j12t, j12s, hbs2 = _level_jobs(m0, m1, m2, eidx)

    y0 = _epi([hbs0, j01s], _N0, params['lin0']['W'], params['lin0']['b'])
    y1 = _epi([j01t, hbs1, j12s], _N1, params['lin1']['W'], params['lin1']['b'])
    y2 = _epi([j12t, hbs2], _N2, params['lin2']['W'], params['lin2']['b'])
    return (y0, y1, y2)


# speedup vs baseline: 3.9986x; 1.0517x over previous
import functools

import jax
import jax.numpy as jnp
from jax import lax
from jax.experimental import pallas as pl
from jax.experimental.pallas import tpu as pltpu
from jax.experimental.pallas import tpu_sc as plsc

_N0, _N1, _N2 = 10000, 40000, 5000
_H = 128
_C = 32
_NS = 0.2
_R0, _R1, _R2 = 10240, 40960, 5120
_NW = 32


def _round_up(x, m):
    return (x + m - 1) // m * m


@functools.cache
def _make_agg(e_pad, n_table, r_out, p_passes):
    F = _H // p_passes
    EBLK = 64 if p_passes == 1 else 128
    EPW = e_pad // _NW
    NB = EPW // EBLK
    assert EPW % EBLK == 0
    rows_pt = r_out // 16
    ZR = 32 if p_passes == 1 else 128
    OB = 64
    assert rows_pt % ZR == 0 and rows_pt % OB == 0

    mesh = plsc.VectorSubcoreMesh(core_axis_name="c", subcore_axis_name="s")

    def body(*refs):
        tabs = refs[:p_passes]
        (gidx, sidx, ag_hbm, as_hbm, acc_out, d_out,
         gbuf, sbuf, wbuf, av1, av2, rows, zrow, obuf, dbuf, wall,
         acc_sh, d_sh, sem, sem1, sem2) = refs[p_passes:]

        cid = lax.axis_index("c")
        sid = lax.axis_index("s")
        wid = cid * 16 + sid
        r0t = sid * rows_pt

        zero16 = jnp.zeros((16,), jnp.float32)
        for j in range(ZR):
            for f in range(F // 16):
                zrow[j, pl.ds(f * 16, 16)] = zero16
        for j in range(rows_pt // 16):
            dbuf[pl.ds(j * 16, 16)] = zero16

        for q in range(p_passes):
            @pl.loop(0, rows_pt // ZR)
            def _zero(k):
                pltpu.sync_copy(zrow, acc_sh.at[pl.ds(r0t + k * ZR, ZR)])
            if q == 0:
                pltpu.sync_copy(dbuf, d_sh.at[pl.ds(r0t, rows_pt)])
            plsc.subcore_barrier()

            @pl.loop(0, NB)
            def _edges(b):
                off = wid * EPW + b * EBLK
                pltpu.sync_copy(gidx.at[pl.ds(off, EBLK)], gbuf)
                pltpu.sync_copy(sidx.at[pl.ds(off, EBLK)], sbuf)
                cp0 = pltpu.async_copy(tabs[q].at[gbuf], rows, sem)
                if q == 0:
                    cp1 = pltpu.async_copy(ag_hbm.at[gbuf], av1, sem1)
                    cp2 = pltpu.async_copy(as_hbm.at[sbuf], av2, sem2)
                    cp1.wait()
                    cp2.wait()
                    for j in range(EBLK // 16):
                        e = (av1[pl.ds(j * 16, 16)] + av2[pl.ds(j * 16, 16)])
                        e = jnp.where(e > 0, e, _NS * e)
                        w16 = jnp.exp(e)
                        wbuf[pl.ds(j * 16, 16)] = w16
                        if p_passes > 1:
                            wall[pl.ds(b * EBLK + j * 16, 16)] = w16
                cp0.wait()
                for j in range(EBLK // 16):
                    if q == 0:
                        w16 = wbuf[pl.ds(j * 16, 16)]
                    else:
                        w16 = wall[pl.ds(b * EBLK + j * 16, 16)]
                    for l in range(16):
                        i = j * 16 + l
                        w = w16[l]
                        for f in range(F // 16):
                            rows[i, pl.ds(f * 16, 16)] = (
                                rows[i, pl.ds(f * 16, 16)] * w)
                if q == 0:
                    pltpu.sync_copy(wbuf, d_sh.at[sbuf], add=True)
                pltpu.sync_copy(rows, acc_sh.at[sbuf], add=True)

            plsc.subcore_barrier()

            @pl.loop(0, rows_pt // OB)
            def _wout(k):
                pltpu.sync_copy(acc_sh.at[pl.ds(r0t + k * OB, OB)], obuf)
                pltpu.sync_copy(obuf,
                                acc_out.at[cid, pl.ds(r0t + k * OB, OB), q])
            if q == 0:
                pltpu.sync_copy(d_sh.at[pl.ds(r0t, rows_pt)], dbuf)
                pltpu.sync_copy(
                    dbuf, d_out.at[pl.ds(cid * r_out + r0t, rows_pt)])
            plsc.subcore_barrier()

    out_type = (jax.ShapeDtypeStruct((2, r_out, p_passes, F), jnp.float32),
                jax.ShapeDtypeStruct((2 * r_out,), jnp.float32))
    scratch = (
        pltpu.VMEM((EBLK,), jnp.int32),
        pltpu.VMEM((EBLK,), jnp.int32),
        pltpu.VMEM((EBLK,), jnp.float32),
        pltpu.VMEM((EBLK,), jnp.float32),
        pltpu.VMEM((EBLK,), jnp.float32),
        pltpu.VMEM((EBLK, F), jnp.float32),
        pltpu.VMEM((ZR, F), jnp.float32),
        pltpu.VMEM((OB, F), jnp.float32),
        pltpu.VMEM((rows_pt,), jnp.float32),
        pltpu.VMEM((EPW if p_passes > 1 else 16,), jnp.float32),
        pltpu.VMEM_SHARED((r_out, F), jnp.float32),
        pltpu.VMEM_SHARED((r_out,), jnp.float32),
        pltpu.SemaphoreType.DMA,
        pltpu.SemaphoreType.DMA,
        pltpu.SemaphoreType.DMA,
    )
    return pl.kernel(body, out_type=out_type, mesh=mesh,
                     scratch_types=scratch,
                     compiler_params=pltpu.CompilerParams(
                         needs_layout_passes=False,
                         use_tc_tiling_on_sc=False))


def _agg(tables, gidx, sidx, ag, as_pad, r_out):
    p = len(tables)
    e_pad = gidx.shape[0]
    fn = _make_agg(e_pad, tables[0].shape[0], r_out, p)
    acc, d = fn(*tables, gidx, sidx, ag, as_pad)
    return acc.reshape(2, r_out, _H), d.reshape(2, r_out)


_BN = 1000


def _proj(x, w):
    n, k = x.shape[0], w.shape[1]

    def kern(x_ref, w_ref, o_ref):
        o_ref[...] = jnp.dot(x_ref[...], w_ref[...],
                             preferred_element_type=jnp.float32)

    return pl.pallas_call(
        kern,
        grid=(n // _BN,),
        in_specs=[pl.BlockSpec((_BN, _H), lambda i: (i, 0)),
                  pl.BlockSpec((_H, k), lambda i: (0, 0))],
        out_specs=pl.BlockSpec((_BN, k), lambda i: (i, 0)),
        out_shape=jax.ShapeDtypeStruct((n, k), jnp.float32),
    )(x, w)


def _epi(jobs, n, w, b=None):
    k = w.shape[1]
    nj = len(jobs)

    def kern(*refs):
        o_ref = refs[-1]
        w_ref = refs[2 * nj]
        parts = None
        for j in range(nj):
            acc = refs[2 * j][...]
            d = refs[2 * j + 1][...]
            dt = d[:, 0] + d[:, 1] + 1e-16
            m = (acc[0] + acc[1]) / dt[:, None]
            parts = m if parts is None else parts + m
        xblk = jnp.maximum(parts, 0.0)
        out = jnp.dot(xblk, w_ref[...], preferred_element_type=jnp.float32)
        if b is not None:
            out = out + refs[2 * nj + 1][...]
        o_ref[...] = out

    in_specs = []
    args = []
    for acc, d in jobs:
        in_specs.append(pl.BlockSpec((2, _BN, _H), lambda i: (0, i, 0)))
        in_specs.append(pl.BlockSpec((_BN, 2), lambda i: (i, 0)))
        args += [acc, d.T]
    in_specs.append(pl.BlockSpec((_H, k), lambda i: (0, 0)))
    args.append(w)
    if b is not None:
        in_specs.append(pl.BlockSpec((1, k), lambda i: (0, 0)))
        args.append(b.reshape(1, k))

    return pl.pallas_call(
        kern,
        grid=(n // _BN,),
        in_specs=in_specs,
        out_specs=pl.BlockSpec((_BN, k), lambda i: (i, 0)),
        out_shape=jax.ShapeDtypeStruct((n, k), jnp.float32),
    )(*args)


def _wcat(lp):
    h0, h1, h2 = lp['hbs0'], lp['hbs1'], lp['hbs2']
    b01, b12 = lp['hbns01'], lp['hbns12']
    z = jnp.zeros
    w0 = jnp.concatenate([
        h0['W'], b01['Ws'],
        (h0['W'] @ h0['a_src'])[:, None], (h0['W'] @ h0['a_dst'])[:, None],
        (b01['Ws'] @ b01['a_s'])[:, None], z((_H, 125))], axis=1)
    w1 = jnp.concatenate([
        b01['Wt'], h1['W'], b12['Ws'],
        (b01['Wt'] @ b01['a_t'])[:, None],
        (h1['W'] @ h1['a_src'])[:, None], (h1['W'] @ h1['a_dst'])[:, None],
        (b12['Ws'] @ b12['a_s'])[:, None], z((_H, 124))], axis=1)
    w2 = jnp.concatenate([
        b12['Wt'], h2['W'],
        (b12['Wt'] @ b12['a_t'])[:, None],
        (h2['W'] @ h2['a_src'])[:, None], (h2['W'] @ h2['a_dst'])[:, None],
        z((_H, 125))], axis=1)
    return w0, w1, w2


def _split(m, p):
    f = _H // p
    return tuple(m[:, f * q:f * (q + 1)] for q in range(p))


def _pad_alpha(a, r):
    return jnp.pad(a, (0, r - a.shape[0]))


def _level_jobs(m0, m1, m2, eidx):
    (g_hbs0, s_hbs0, g_01t, s_01t, g_01s, s_01s, g_hbs1, s_hbs1,
     g_12t, s_12t, g_12s, s_12s, g_hbs2, s_hbs2) = eidx

    a_src0, a_dst0, a_s01 = m0[:, 256], m0[:, 257], m0[:, 258]
    a_t01, a_src1, a_dst1, a_s12 = (m1[:, 384], m1[:, 385], m1[:, 386],
                                    m1[:, 387])
    a_t12, a_src2, a_dst2 = m2[:, 256], m2[:, 257], m2[:, 258]

    hbs0 = _agg(_split(m0[:, :128], 2), g_hbs0, s_hbs0,
                a_src0, _pad_alpha(a_dst0, _R0), _R0)
    j01t = _agg(_split(m0[:, 128:256], 8), g_01t, s_01t,
                a_s01, _pad_alpha(a_t01, _R1), _R1)
    j01s = _agg(_split(m1[:, :128], 2), g_01s, s_01s,
                a_t01, _pad_alpha(a_s01, _R0), _R0)
    hbs1 = _agg(_split(m1[:, 128:256], 8), g_hbs1, s_hbs1,
                a_src1, _pad_alpha(a_dst1, _R1), _R1)
    j12t = _agg((m1[:, 256:384],), g_12t, s_12t,
                a_s12, _pad_alpha(a_t12, _R2), _R2)
    j12s = _agg(_split(m2[:, :128], 8), g_12s, s_12s,
                a_t12, _pad_alpha(a_s12, _R1), _R1)
    hbs2 = _agg((m2[:, 128:256],), g_hbs2, s_hbs2,
                a_src2, _pad_alpha(a_dst2, _R2), _R2)
    return hbs0, j01t, j01s, hbs1, j12t, j12s, hbs2


def _pad_edges(g, s, n_t, unit):
    e = g.shape[0]
    e_pad = _round_up(e, unit)
    return (jnp.pad(g, (0, e_pad - e)),
            jnp.pad(s, (0, e_pad - e), constant_values=n_t))


def kernel(x_0, x_1, x_2, adjacency_0, adjacency_1, adjacency_2,
           incidence_1, incidence_2, params):
    u1, u4 = 2048, 4096
    g_hbs0, s_hbs0 = _pad_edges(adjacency_0[0], adjacency_0[1], _N0, u4)
    g_01t, s_01t = _pad_edges(incidence_1[0], incidence_1[1], _N1, u4)
    g_01s, s_01s = _pad_edges(incidence_1[1], incidence_1[0], _N0, u4)
    g_hbs1, s_hbs1 = _pad_edges(adjacency_1[0], adjacency_1[1], _N1, u4)
    g_12t, s_12t = _pad_edges(incidence_2[0], incidence_2[1], _N2, u1)
    g_12s, s_12s = _pad_edges(incidence_2[1], incidence_2[0], _N1, u4)
    g_hbs2, s_hbs2 = _pad_edges(adjacency_2[0], adjacency_2[1], _N2, u1)
    eidx = (g_hbs0, s_hbs0, g_01t, s_01t, g_01s, s_01s, g_hbs1, s_hbs1,
            g_12t, s_12t, g_12s, s_12s, g_hbs2, s_hbs2)

    w0_l1, w1_l1, w2_l1 = _wcat(params['l1'])
    w0_l2, w1_l2, w2_l2 = _wcat(params['l2'])

    m0 = _proj(x_0, w0_l1)
    m1 = _proj(x_1, w1_l1)
    m2 = _proj(x_2, w2_l1)
    hbs0, j01t, j01s, hbs1, j12t, j12s, hbs2 = _level_jobs(m0, m1, m2, eidx)

    m0 = _epi([hbs0, j01s], _N0, w0_l2)
    m1 = _epi([j01t, hbs1, j12s], _N1, w1_l2)
    m2 = _epi([j12t, hbs2], _N2, w2_l2)

    hbs0, j01t, j01s, hbs1, j12t, j12s, hbs2 = _level_jobs(m0, m1, m2, eidx)

    y0 = _epi([hbs0, j01s], _N0, params['lin0']['W'], params['lin0']['b'])
    y1 = _epi([j01t, hbs1, j12s], _N1, params['lin1']['W'], params['lin1']['b'])
    y2 = _epi([j12t, hbs2], _N2, params['lin2']['W'], params['lin2']['b'])
    return (y0, y1, y2)


# speedup vs baseline: 5.8360x; 1.4595x over previous
import functools

import jax
import jax.numpy as jnp
from jax import lax
from jax.experimental import pallas as pl
from jax.experimental.pallas import tpu as pltpu
from jax.experimental.pallas import tpu_sc as plsc

_N0, _N1, _N2 = 10000, 40000, 5000
_H = 128
_C = 32
_NS = 0.2
_R0, _R1, _R2 = 10240, 40960, 5120
_NW = 32


def _round_up(x, m):
    return (x + m - 1) // m * m


@functools.cache
def _make_agg(e_pad, n_table, r_out, p_passes):
    F = _H // p_passes
    EBLK = 64 if p_passes == 1 else 128
    EPW = e_pad // _NW
    NB = EPW // EBLK
    assert EPW % EBLK == 0
    rows_pt = r_out // 16
    ZR = 32 if p_passes == 1 else 128
    OB = 64
    assert rows_pt % ZR == 0 and rows_pt % OB == 0

    mesh = plsc.VectorSubcoreMesh(core_axis_name="c", subcore_axis_name="s")

    def body(*refs):
        tabs = refs[:p_passes]
        (gidx, sidx, ag_hbm, as_hbm, acc_out, d_out,
         gbuf, sbuf, wbuf, av1, av2, rowsA, rowsB, zrow, obuf, dbuf, wall,
         acc_sh, d_sh, semig, semis, semr, sema1, sema2, semw, semd
         ) = refs[p_passes:]

        cid = lax.axis_index("c")
        sid = lax.axis_index("s")
        wid = cid * 16 + sid
        r0t = sid * rows_pt

        zero16 = jnp.zeros((16,), jnp.float32)
        for j in range(ZR):
            for f in range(F // 16):
                zrow[j, pl.ds(f * 16, 16)] = zero16
        for j in range(rows_pt // 16):
            dbuf[pl.ds(j * 16, 16)] = zero16

        def idx_issue(b):
            off = wid * EPW + b * EBLK
            s = b & 3
            pltpu.async_copy(gidx.at[pl.ds(off, EBLK)], gbuf.at[s], semig)
            pltpu.async_copy(sidx.at[pl.ds(off, EBLK)], sbuf.at[s], semis)

        def idx_wait(b):
            off = wid * EPW + b * EBLK
            s = b & 3
            pltpu.make_async_copy(gidx.at[pl.ds(off, EBLK)], gbuf.at[s],
                                  semig).wait()
            pltpu.make_async_copy(sidx.at[pl.ds(off, EBLK)], sbuf.at[s],
                                  semis).wait()

        def gather_issue(b, q):
            p, s = b & 1, b & 3
            pltpu.async_copy(tabs[q].at[gbuf.at[s]], rowsA.at[p], semr)
            if q == 0:
                pltpu.async_copy(ag_hbm.at[gbuf.at[s]], av1.at[p], sema1)
                pltpu.async_copy(as_hbm.at[sbuf.at[s]], av2.at[p], sema2)

        def gather_wait(b, q):
            p, s = b & 1, b & 3
            pltpu.make_async_copy(tabs[q].at[gbuf.at[s]], rowsA.at[p],
                                  semr).wait()
            if q == 0:
                pltpu.make_async_copy(ag_hbm.at[gbuf.at[s]], av1.at[p],
                                      sema1).wait()
                pltpu.make_async_copy(as_hbm.at[sbuf.at[s]], av2.at[p],
                                      sema2).wait()

        def scatter_issue(b, q):
            p, s = b & 1, b & 3
            pltpu.async_copy(rowsB.at[p], acc_sh.at[sbuf.at[s]], semw,
                             add=True)
            if q == 0:
                pltpu.async_copy(wbuf.at[p], d_sh.at[sbuf.at[s]], semd,
                                 add=True)

        def scatter_wait(b, q):
            p, s = b & 1, b & 3
            pltpu.make_async_copy(rowsB.at[p], acc_sh.at[sbuf.at[s]],
                                  semw).wait()
            if q == 0:
                pltpu.make_async_copy(wbuf.at[p], d_sh.at[sbuf.at[s]],
                                      semd).wait()

        def compute(b, q):
            p = b & 1
            if q == 0:
                for j in range(EBLK // 16):
                    e = (av1[p, pl.ds(j * 16, 16)]
                         + av2[p, pl.ds(j * 16, 16)])
                    e = jnp.where(e > 0, e, _NS * e)
                    w16 = jnp.exp(e)
                    wbuf[p, pl.ds(j * 16, 16)] = w16
                    if p_passes > 1:
                        wall[pl.ds(b * EBLK + j * 16, 16)] = w16
            for j in range(EBLK // 16):
                if q == 0:
                    w16 = wbuf[p, pl.ds(j * 16, 16)]
                else:
                    w16 = wall[pl.ds(b * EBLK + j * 16, 16)]
                for l in range(16):
                    i = j * 16 + l
                    w = w16[l]
                    for f in range(F // 16):
                        rowsB[p, i, pl.ds(f * 16, 16)] = (
                            rowsA[p, i, pl.ds(f * 16, 16)] * w)

        for q in range(p_passes):
            @pl.loop(0, rows_pt // ZR)
            def _zero(k):
                pltpu.sync_copy(zrow, acc_sh.at[pl.ds(r0t + k * ZR, ZR)])
            if q == 0:
                pltpu.sync_copy(dbuf, d_sh.at[pl.ds(r0t, rows_pt)])
            plsc.subcore_barrier()

            off0 = wid * EPW
            pltpu.sync_copy(gidx.at[pl.ds(off0, EBLK)], gbuf.at[0])
            pltpu.sync_copy(sidx.at[pl.ds(off0, EBLK)], sbuf.at[0])
            gather_issue(0, q)
            idx_issue(1)

            @pl.loop(0, NB)
            def _edges(b):
                @pl.when(b >= 2)
                def _():
                    scatter_wait(b - 2, q)

                @pl.when(b + 2 < NB)
                def _():
                    idx_issue(b + 2)

                @pl.when(b + 1 < NB)
                def _():
                    idx_wait(b + 1)
                    gather_issue(b + 1, q)

                gather_wait(b, q)
                compute(b, q)
                scatter_issue(b, q)

            scatter_wait(NB - 2, q)
            scatter_wait(NB - 1, q)
            plsc.subcore_barrier()

            @pl.loop(0, rows_pt // OB)
            def _wout(k):
                pltpu.sync_copy(acc_sh.at[pl.ds(r0t + k * OB, OB)], obuf)
                pltpu.sync_copy(obuf,
                                acc_out.at[cid, pl.ds(r0t + k * OB, OB), q])
            if q == 0:
                pltpu.sync_copy(d_sh.at[pl.ds(r0t, rows_pt)], dbuf)
                pltpu.sync_copy(
                    dbuf, d_out.at[pl.ds(cid * r_out + r0t, rows_pt)])
            plsc.subcore_barrier()

    out_type = (jax.ShapeDtypeStruct((2, r_out, p_passes, F), jnp.float32),
                jax.ShapeDtypeStruct((2 * r_out,), jnp.float32))
    scratch = (
        pltpu.VMEM((4, EBLK), jnp.int32),
        pltpu.VMEM((4, EBLK), jnp.int32),
        pltpu.VMEM((2, EBLK), jnp.float32),
        pltpu.VMEM((2, EBLK), jnp.float32),
        pltpu.VMEM((2, EBLK), jnp.float32),
        pltpu.VMEM((2, EBLK, F), jnp.float32),
        pltpu.VMEM((2, EBLK, F), jnp.float32),
        pltpu.VMEM((ZR, F), jnp.float32),
        pltpu.VMEM((OB, F), jnp.float32),
        pltpu.VMEM((rows_pt,), jnp.float32),
        pltpu.VMEM((EPW if p_passes > 1 else 16,), jnp.float32),
        pltpu.VMEM_SHARED((r_out, F), jnp.float32),
        pltpu.VMEM_SHARED((r_out,), jnp.float32),
        pltpu.SemaphoreType.DMA,
        pltpu.SemaphoreType.DMA,
        pltpu.SemaphoreType.DMA,
        pltpu.SemaphoreType.DMA,
        pltpu.SemaphoreType.DMA,
        pltpu.SemaphoreType.DMA,
        pltpu.SemaphoreType.DMA,
    )
    return pl.kernel(body, out_type=out_type, mesh=mesh,
                     scratch_types=scratch,
                     compiler_params=pltpu.CompilerParams(
                         needs_layout_passes=False,
                         use_tc_tiling_on_sc=False))


def _agg(tables, gidx, sidx, ag, as_pad, r_out):
    p = len(tables)
    e_pad = gidx.shape[0]
    fn = _make_agg(e_pad, tables[0].shape[0], r_out, p)
    acc, d = fn(*tables, gidx, sidx, ag, as_pad)
    return acc.reshape(2, r_out, _H), d.reshape(2, r_out)


_BN = 1000


def _proj(x, w):
    n, k = x.shape[0], w.shape[1]

    def kern(x_ref, w_ref, o_ref):
        o_ref[...] = jnp.dot(x_ref[...], w_ref[...],
                             preferred_element_type=jnp.float32)

    return pl.pallas_call(
        kern,
        grid=(n // _BN,),
        in_specs=[pl.BlockSpec((_BN, _H), lambda i: (i, 0)),
                  pl.BlockSpec((_H, k), lambda i: (0, 0))],
        out_specs=pl.BlockSpec((_BN, k), lambda i: (i, 0)),
        out_shape=jax.ShapeDtypeStruct((n, k), jnp.float32),
    )(x, w)


def _epi(jobs, n, w, b=None):
    k = w.shape[1]
    nj = len(jobs)

    def kern(*refs):
        o_ref = refs[-1]
        w_ref = refs[2 * nj]
        parts = None
        for j in range(nj):
            acc = refs[2 * j][...]
            d = refs[2 * j + 1][...]
            dt = d[:, 0] + d[:, 1] + 1e-16
            m = (acc[0] + acc[1]) / dt[:, None]
            parts = m if parts is None else parts + m
        xblk = jnp.maximum(parts, 0.0)
        out = jnp.dot(xblk, w_ref[...], preferred_element_type=jnp.float32)
        if b is not None:
            out = out + refs[2 * nj + 1][...]
        o_ref[...] = out

    in_specs = []
    args = []
    for acc, d in jobs:
        in_specs.append(pl.BlockSpec((2, _BN, _H), lambda i: (0, i, 0)))
        in_specs.append(pl.BlockSpec((_BN, 2), lambda i: (i, 0)))
        args += [acc, d.T]
    in_specs.append(pl.BlockSpec((_H, k), lambda i: (0, 0)))
    args.append(w)
    if b is not None:
        in_specs.append(pl.BlockSpec((1, k), lambda i: (0, 0)))
        args.append(b.reshape(1, k))

    return pl.pallas_call(
        kern,
        grid=(n // _BN,),
        in_specs=in_specs,
        out_specs=pl.BlockSpec((_BN, k), lambda i: (i, 0)),
        out_shape=jax.ShapeDtypeStruct((n, k), jnp.float32),
    )(*args)


def _wcat(lp):
    h0, h1, h2 = lp['hbs0'], lp['hbs1'], lp['hbs2']
    b01, b12 = lp['hbns01'], lp['hbns12']
    z = jnp.zeros
    w0 = jnp.concatenate([
        h0['W'], b01['Ws'],
        (h0['W'] @ h0['a_src'])[:, None], (h0['W'] @ h0['a_dst'])[:, None],
        (b01['Ws'] @ b01['a_s'])[:, None], z((_H, 125))], axis=1)
    w1 = jnp.concatenate([
        b01['Wt'], h1['W'], b12['Ws'],
        (b01['Wt'] @ b01['a_t'])[:, None],
        (h1['W'] @ h1['a_src'])[:, None], (h1['W'] @ h1['a_dst'])[:, None],
        (b12['Ws'] @ b12['a_s'])[:, None], z((_H, 124))], axis=1)
    w2 = jnp.concatenate([
        b12['Wt'], h2['W'],
        (b12['Wt'] @ b12['a_t'])[:, None],
        (h2['W'] @ h2['a_src'])[:, None], (h2['W'] @ h2['a_dst'])[:, None],
        z((_H, 125))], axis=1)
    return w0, w1, w2


def _split(m, p):
    f = _H // p
    return tuple(m[:, f * q:f * (q + 1)] for q in range(p))


def _pad_alpha(a, r):
    return jnp.pad(a, (0, r - a.shape[0]))


def _level_jobs(m0, m1, m2, eidx):
    (g_hbs0, s_hbs0, g_01t, s_01t, g_01s, s_01s, g_hbs1, s_hbs1,
     g_12t, s_12t, g_12s, s_12s, g_hbs2, s_hbs2) = eidx

    a_src0, a_dst0, a_s01 = m0[:, 256], m0[:, 257], m0[:, 258]
    a_t01, a_src1, a_dst1, a_s12 = (m1[:, 384], m1[:, 385], m1[:, 386],
                                    m1[:, 387])
    a_t12, a_src2, a_dst2 = m2[:, 256], m2[:, 257], m2[:, 258]

    hbs0 = _agg(_split(m0[:, :128], 2), g_hbs0, s_hbs0,
                a_src0, _pad_alpha(a_dst0, _R0), _R0)
    j01t = _agg(_split(m0[:, 128:256], 8), g_01t, s_01t,
                a_s01, _pad_alpha(a_t01, _R1), _R1)
    j01s = _agg(_split(m1[:, :128], 2), g_01s, s_01s,
                a_t01, _pad_alpha(a_s01, _R0), _R0)
    hbs1 = _agg(_split(m1[:, 128:256], 8), g_hbs1, s_hbs1,
                a_src1, _pad_alpha(a_dst1, _R1), _R1)
    j12t = _agg((m1[:, 256:384],), g_12t, s_12t,
                a_s12, _pad_alpha(a_t12, _R2), _R2)
    j12s = _agg(_split(m2[:, :128], 8), g_12s, s_12s,
                a_t12, _pad_alpha(a_s12, _R1), _R1)
    hbs2 = _agg((m2[:, 128:256],), g_hbs2, s_hbs2,
                a_src2, _pad_alpha(a_dst2, _R2), _R2)
    return hbs0, j01t, j01s, hbs1, j12t, j12s, hbs2


def _pad_edges(g, s, n_t, unit):
    e = g.shape[0]
    e_pad = _round_up(e, unit)
    return (jnp.pad(g, (0, e_pad - e)),
            jnp.pad(s, (0, e_pad - e), constant_values=n_t))


def kernel(x_0, x_1, x_2, adjacency_0, adjacency_1, adjacency_2,
           incidence_1, incidence_2, params):
    u1, u4 = 2048, 4096
    g_hbs0, s_hbs0 = _pad_edges(adjacency_0[0], adjacency_0[1], _N0, u4)
    g_01t, s_01t = _pad_edges(incidence_1[0], incidence_1[1], _N1, u4)
    g_01s, s_01s = _pad_edges(incidence_1[1], incidence_1[0], _N0, u4)
    g_hbs1, s_hbs1 = _pad_edges(adjacency_1[0], adjacency_1[1], _N1, u4)
    g_12t, s_12t = _pad_edges(incidence_2[0], incidence_2[1], _N2, u1)
    g_12s, s_12s = _pad_edges(incidence_2[1], incidence_2[0], _N1, u4)
    g_hbs2, s_hbs2 = _pad_edges(adjacency_2[0], adjacency_2[1], _N2, u1)
    eidx = (g_hbs0, s_hbs0, g_01t, s_01t, g_01s, s_01s, g_hbs1, s_hbs1,
            g_12t, s_12t, g_12s, s_12s, g_hbs2, s_hbs2)

    w0_l1, w1_l1, w2_l1 = _wcat(params['l1'])
    w0_l2, w1_l2, w2_l2 = _wcat(params['l2'])

    m0 = _proj(x_0, w0_l1)
    m1 = _proj(x_1, w1_l1)
    m2 = _proj(x_2, w2_l1)
    hbs0, j01t, j01s, hbs1, j12t, j12s, hbs2 = _level_jobs(m0, m1, m2, eidx)

    m0 = _epi([hbs0, j01s], _N0, w0_l2)
    m1 = _epi([j01t, hbs1, j12s], _N1, w1_l2)
    m2 = _epi([j12t, hbs2], _N2, w2_l2)

    hbs0, j01t, j01s, hbs1, j12t, j12s, hbs2 = _level_jobs(m0, m1, m2, eidx)

    y0 = _epi([hbs0, j01s], _N0, params['lin0']['W'], params['lin0']['b'])
    y1 = _epi([j01t, hbs1, j12s], _N1, params['lin1']['W'], params['lin1']['b'])
    y2 = _epi([j12t, hbs2], _N2, params['lin2']['W'], params['lin2']['b'])
    return (y0, y1, y2)
